# baseline (speedup 1.0000x reference)
import functools

import jax
import jax.numpy as jnp
from jax import lax
from jax.experimental import pallas as pl
from jax.experimental.pallas import tpu as pltpu
from jax.experimental.pallas import tpu_sc as plsc

N = 10000
D_IN = 128
D_H = 256
D_OUT = 128
E = 320000

N_PAD = 10240
PAD_NODE = N
E_ROWS = 2560
E_PAD = E_ROWS * 128

NC = 2
NS = 16
ROWS_W = E_ROWS // (NC * NS)
ROWS_T = E_ROWS // NS
ROWS_SC = 40
BATCH = 32
NBUF = 8
NSC = ROWS_T // ROWS_SC
ROWS_OUT = N_PAD // NS

_mesh = plsc.VectorSubcoreMesh(core_axis_name="c", subcore_axis_name="s")
_sc_params = pltpu.CompilerParams(needs_layout_passes=False)


@functools.partial(
    pl.kernel,
    out_type=jax.ShapeDtypeStruct((NC * NS, N_PAD), jnp.float32),
    mesh=_mesh,
    compiler_params=_sc_params,
    scratch_types=[
        pltpu.VMEM((ROWS_W, 128), jnp.int32),
        pltpu.VMEM((N_PAD,), jnp.float32),
    ],
)
def _deg_kernel(dst_hbm, out_hbm, idx_v, hist_v):
    c = lax.axis_index("c")
    s = lax.axis_index("s")
    w = c * NS + s

    zero16 = jnp.zeros((16,), jnp.float32)

    def zbody(i, carry):
        hist_v[pl.ds(i * 16, 16)] = zero16
        return carry

    lax.fori_loop(0, N_PAD // 16, zbody, 0)

    pltpu.sync_copy(dst_hbm.at[pl.ds(w * ROWS_W, ROWS_W)], idx_v)

    ones16 = jnp.ones((16,), jnp.float32)

    def ebody(i, carry):
        r = i // 8
        j = i % 8
        iv = idx_v[r, pl.ds(j * 16, 16)]
        plsc.addupdate_scatter(hist_v, [iv], ones16)
        return carry

    lax.fori_loop(0, ROWS_W * 8, ebody, 0)

    pltpu.sync_copy(hist_v, out_hbm.at[w])


def _make_scatter():
    D = 128
    rows_t = E_ROWS // (NC * NS)
    nsc = rows_t // ROWS_SC

    @functools.partial(
        pl.kernel,
        out_type=(
            jax.ShapeDtypeStruct((N_PAD, D), jnp.float32),
            jax.ShapeDtypeStruct((N_PAD, D), jnp.float32),
        ),
        mesh=_mesh,
        compiler_params=_sc_params,
        scratch_types=[
            pltpu.VMEM((ROWS_SC, 128), jnp.int32),
            pltpu.VMEM((ROWS_SC, 128), jnp.int32),
            pltpu.VMEM((NBUF, BATCH, D), jnp.float32),
            pltpu.VMEM_SHARED((N_PAD, D), jnp.float32),
        ] + [pltpu.SemaphoreType.DMA] * (2 * NBUF),
    )
    def scat(ytab, src_hbm, dst_hbm, o0, o1,
             src_v, dst_v, bufs, acc_sh, *sems):
        c = lax.axis_index("c")
        s = lax.axis_index("s")
        gsems = sems[:NBUF]
        ssems = sems[NBUF:]

        rb = (c * NS + s) * rows_t

        zero16 = jnp.zeros((16,), jnp.float32)

        def zbody(i, carry):
            r = i // (D // 16)
            j = i % (D // 16)
            bufs[0, r, pl.ds(j * 16, 16)] = zero16
            return carry

        lax.fori_loop(0, BATCH * (D // 16), zbody, 0)
        ob = s * ROWS_OUT
        for k in range(ROWS_OUT // BATCH):
            pltpu.sync_copy(bufs.at[0],
                            acc_sh.at[pl.ds(ob + k * BATCH, BATCH)])
        plsc.subcore_barrier()

        def body(ytab, otab):
            def gidx(r, h):
                return src_v.at[r, pl.ds(h * BATCH, BATCH)]

            def didx(r, h):
                return dst_v.at[r, pl.ds(h * BATCH, BATCH)]

            def fire_gather(r, h, j):
                pltpu.async_copy(ytab.at[gidx(r, h)], bufs.at[j], gsems[j])

            def wait_gather(r, h, j):
                pltpu.make_async_copy(
                    ytab.at[gidx(r, h)], bufs.at[j], gsems[j]).wait()

            def fire_scatter(r, h, j):
                pltpu.async_copy(
                    bufs.at[j], acc_sh.at[didx(r, h)], ssems[j], add=True)

            def wait_scatter(r, h, j):
                pltpu.make_async_copy(
                    bufs.at[j], acc_sh.at[didx(r, h)], ssems[j]).wait()

            BPR = 128 // BATCH
            RPG = NBUF // BPR
            L = NBUF // 2
            nq = ROWS_SC // RPG

            def group(q, carry):
                for m in range(NBUF):
                    r = RPG * q + m // BPR
                    h = m % BPR
                    wait_gather(r, h, m)
                    fire_scatter(r, h, m)
                    jn = (m + L) % NBUF
                    r2 = RPG * q + (m - L) // BPR
                    h2 = (m - L) % BPR
                    rn = RPG * q + (m + L) // BPR
                    hn = (m + L) % BPR
                    if m < L:
                        @pl.when(q > 0)
                        def _():
                            wait_scatter(r2, h2, jn)
                        fire_gather(rn, hn, jn)
                    else:
                        wait_scatter(r2, h2, jn)

                        @pl.when(q < nq - 1)
                        def _():
                            fire_gather(rn, hn, jn)
                return carry

            def chunk_body(ci, carry):
                rbase = rb + ci * ROWS_SC
                cp1 = pltpu.async_copy(
                    src_hbm.at[pl.ds(rbase, ROWS_SC)], src_v, gsems[0])
                cp2 = pltpu.async_copy(
                    dst_hbm.at[pl.ds(rbase, ROWS_SC)], dst_v, gsems[1])
                cp1.wait()
                cp2.wait()
                for b in range(L):
                    fire_gather(b // BPR, b % BPR, b)
                lax.fori_loop(0, nq, group, 0)
                nb = ROWS_SC * BPR
                for b in range(nb - L, nb):
                    wait_scatter(b // BPR, b % BPR, b % NBUF)
                return carry

            lax.fori_loop(0, nsc, chunk_body, 0)

            plsc.subcore_barrier()
            for k in range(ROWS_OUT // 128):
                pltpu.sync_copy(acc_sh.at[pl.ds(ob + k * 128, 128)],
                                otab.at[pl.ds(ob + k * 128, 128)])

        @pl.when(c == 0)
        def _():
            body(ytab, o0)

        @pl.when(c == 1)
        def _():
            body(ytab, o1)

    return scat


_scatter = _make_scatter()


_BLK = 2048
_GRID = N_PAD // _BLK


def _dis_block(pt):
    deg = jnp.sum(pt, axis=1, keepdims=True) + 1.0
    return lax.rsqrt(deg)


def _tca_body(x_ref, pt_ref, xs_ref):
    xs_ref[...] = x_ref[...] * _dis_block(pt_ref[...])


def _tcb_body(a0_ref, a1_ref, xs_ref, pt_ref, w1_ref, b1_ref, w2_ref,
              y2_ref):
    dis = _dis_block(pt_ref[...])
    mx = (a0_ref[...] + a1_ref[...] + xs_ref[...]) * dis
    h = jnp.maximum(
        jnp.dot(mx, w1_ref[...], preferred_element_type=jnp.float32)
        + b1_ref[...], 0.0)
    y2_ref[...] = jnp.dot(h, w2_ref[...],
                          preferred_element_type=jnp.float32) * dis


def _tcc_body(a0_ref, a1_ref, y2_ref, pt_ref, b2_ref, out_ref):
    dis = _dis_block(pt_ref[...])
    pre = a0_ref[...] + a1_ref[...] + y2_ref[...]
    out_ref[...] = jnp.maximum(pre * dis + b2_ref[...], 0.0)


def _row_spec(d):
    return pl.BlockSpec((_BLK, d), lambda i: (i, 0))


def _full_spec(r, d):
    return pl.BlockSpec((r, d), lambda i: (0, 0))


_tca = pl.pallas_call(
    _tca_body,
    grid=(_GRID,),
    in_specs=[_row_spec(D_IN), _row_spec(NC * NS)],
    out_specs=_row_spec(D_IN),
    out_shape=jax.ShapeDtypeStruct((N_PAD, D_IN), jnp.float32),
)

_tcb = pl.pallas_call(
    _tcb_body,
    grid=(_GRID,),
    in_specs=[_row_spec(D_IN), _row_spec(D_IN), _row_spec(D_IN),
              _row_spec(NC * NS), _full_spec(D_IN, D_H), _full_spec(1, D_H),
              _full_spec(D_H, D_OUT)],
    out_specs=_row_spec(D_OUT),
    out_shape=jax.ShapeDtypeStruct((N_PAD, D_OUT), jnp.float32),
)

_tcc = pl.pallas_call(
    _tcc_body,
    grid=(_GRID,),
    in_specs=[_row_spec(D_OUT), _row_spec(D_OUT), _row_spec(D_OUT),
              _row_spec(NC * NS), _full_spec(1, D_OUT)],
    out_specs=_row_spec(D_OUT),
    out_shape=jax.ShapeDtypeStruct((N_PAD, D_OUT), jnp.float32),
)


def kernel(x, edge_index, W1, b1, W2, b2):
    ei = edge_index.astype(jnp.int32)
    pad = PAD_NODE + (jnp.arange(E_PAD - E, dtype=jnp.int32) % (N_PAD - N))
    src2d = jnp.concatenate([ei[0], pad]).reshape(E_ROWS, 128)
    dst2d = jnp.concatenate([ei[1], pad]).reshape(E_ROWS, 128)
    x_pad = jnp.pad(x, ((0, N_PAD - N), (0, 0)))

    partials = _deg_kernel(dst2d)
    pt = partials.T

    xs = _tca(x_pad, pt)
    a1_0, a1_1 = _scatter(xs, src2d, dst2d)
    y2 = _tcb(a1_0, a1_1, xs, pt, W1, b1.reshape(1, D_H), W2)
    a2_0, a2_1 = _scatter(y2, src2d, dst2d)
    out = _tcc(a2_0, a2_1, y2, pt, b2.reshape(1, D_OUT))
    return out[:N]
